# baseline (speedup 1.0000x reference)
import functools

import jax
import jax.numpy as jnp
from jax import lax
from jax.experimental import pallas as pl
from jax.experimental.pallas import tpu as pltpu
from jax.experimental.pallas import tpu_sc as plsc

BATCH = 4096
INPUT_DIM = 39
N_CONT = 13
N_CAT = 26
VOCAB = 100000
EMB = 32
LANES = 16
OUT_DIM = N_CONT + N_CAT * EMB
SPLIT = 13

NUM_CORES = 2
NUM_SUBCORES = 16
NUM_WORKERS = NUM_CORES * NUM_SUBCORES
B_PER_W = BATCH // NUM_WORKERS
VREGS_PER_TABLE = B_PER_W // LANES

_mesh = plsc.VectorSubcoreMesh(
    core_axis_name="c", subcore_axis_name="s",
    num_cores=NUM_CORES, num_subcores=NUM_SUBCORES,
)


def _make_half_kernel(n_tab: int, xt_row0: int, with_cont: bool):
    head = N_CONT if with_cont else 0
    out_rows = head + n_tab * EMB

    @functools.partial(
        pl.kernel,
        out_type=jax.ShapeDtypeStruct((out_rows, BATCH), jnp.float32),
        mesh=_mesh,
        compiler_params=pltpu.CompilerParams(
            use_tc_tiling_on_sc=False, needs_layout_passes=False),
        scratch_types=[
            pltpu.VMEM((n_tab, B_PER_W), jnp.float32),
            pltpu.VMEM((out_rows, B_PER_W), jnp.float32),
            pltpu.SemaphoreType.DMA,
        ],
    )
    def half_kernel(tt_hbm, xt_hbm, out_hbm, xi_v, asm_v, sem):
        wid = lax.axis_index("s") * NUM_CORES + lax.axis_index("c")
        base_b = wid * B_PER_W

        pltpu.sync_copy(
            xt_hbm.at[pl.ds(xt_row0, n_tab), pl.ds(base_b, B_PER_W)], xi_v)
        if with_cont:
            pltpu.sync_copy(
                xt_hbm.at[pl.ds(0, N_CONT), pl.ds(base_b, B_PER_W)],
                asm_v.at[pl.ds(0, N_CONT), :])

        def table_bytes_wait():
            pltpu.make_async_copy(
                xt_hbm.at[pl.ds(0, EMB), pl.ds(0, B_PER_W)],
                asm_v.at[pl.ds(head, EMB), :],
                sem).wait()

        def per_table(j, carry):
            vjs = [
                xi_v[j, pl.ds(h * LANES, LANES)].astype(jnp.int32)
                for h in range(VREGS_PER_TABLE)
            ]
            row0 = head + j * EMB
            for e in range(EMB):
                base = (j * EMB + e) * VOCAB
                for h in range(VREGS_PER_TABLE):
                    pltpu.async_copy(
                        tt_hbm.at[vjs[h] + base],
                        asm_v.at[row0 + e, pl.ds(h * LANES, LANES)],
                        sem)

            @pl.when(j > 0)
            def _():
                table_bytes_wait()

            return carry

        lax.fori_loop(0, n_tab, per_table, 0)
        table_bytes_wait()

        pltpu.sync_copy(asm_v, out_hbm.at[:, pl.ds(base_b, B_PER_W)])

    return half_kernel


_kernel_a = _make_half_kernel(SPLIT, N_CONT, True)
_kernel_b = _make_half_kernel(N_CAT - SPLIT, N_CONT + SPLIT, False)


def kernel(x, tables):
    xt = x.T
    tt_a = jnp.transpose(tables[:SPLIT], (0, 2, 1)).reshape(-1)
    tt_b = jnp.transpose(tables[SPLIT:], (0, 2, 1)).reshape(-1)
    out_a = _kernel_a(tt_a, xt)
    out_b = _kernel_b(tt_b, xt)
    return jnp.concatenate([out_a, out_b], axis=0).T


# speedup vs baseline: 1.2381x; 1.2381x over previous
import functools

import jax
import jax.numpy as jnp
from jax import lax
from jax.experimental import pallas as pl
from jax.experimental.pallas import tpu as pltpu
from jax.experimental.pallas import tpu_sc as plsc

BATCH = 4096
INPUT_DIM = 39
N_CONT = 13
N_CAT = 26
VOCAB = 100000
EMB = 32
LANES = 16
OUT_DIM = N_CONT + N_CAT * EMB

NUM_CORES = 2
NUM_SUBCORES = 16
NUM_WORKERS = NUM_CORES * NUM_SUBCORES
B_PER_W = BATCH // NUM_WORKERS
VREGS_PER_TABLE = B_PER_W // LANES

_mesh = plsc.VectorSubcoreMesh(
    core_axis_name="c", subcore_axis_name="s",
    num_cores=NUM_CORES, num_subcores=NUM_SUBCORES,
)


@functools.partial(
    pl.kernel,
    out_type=jax.ShapeDtypeStruct((OUT_DIM, BATCH), jnp.float32),
    mesh=_mesh,
    compiler_params=pltpu.CompilerParams(
        use_tc_tiling_on_sc=False, needs_layout_passes=False),
    scratch_types=[
        pltpu.VMEM((N_CAT, B_PER_W), jnp.float32),
        pltpu.VMEM((OUT_DIM, B_PER_W), jnp.float32),
        pltpu.SemaphoreType.DMA,
    ],
)
def _emb_kernel(tt_hbm, xt_hbm, out_hbm, xi_v, asm_v, sem):
    wid = lax.axis_index("s") * NUM_CORES + lax.axis_index("c")
    base_b = wid * B_PER_W

    pltpu.sync_copy(
        xt_hbm.at[pl.ds(N_CONT, N_CAT), pl.ds(base_b, B_PER_W)], xi_v)
    pltpu.sync_copy(xt_hbm.at[pl.ds(0, N_CONT), pl.ds(base_b, B_PER_W)],
                    asm_v.at[pl.ds(0, N_CONT), :])

    def table_bytes_wait():
        pltpu.make_async_copy(
            xt_hbm.at[pl.ds(0, EMB), pl.ds(0, B_PER_W)],
            asm_v.at[pl.ds(N_CONT, EMB), :],
            sem).wait()

    def per_table(j, carry):
        vjs = [
            xi_v[j, pl.ds(h * LANES, LANES)].astype(jnp.int32)
            for h in range(VREGS_PER_TABLE)
        ]
        row0 = N_CONT + j * EMB
        for e in range(EMB):
            base = (j * EMB + e) * VOCAB
            for h in range(VREGS_PER_TABLE):
                flat_idx = vjs[h] + base
                pltpu.async_copy(
                    tt_hbm.at[flat_idx],
                    asm_v.at[row0 + e, pl.ds(h * LANES, LANES)],
                    sem)

        @pl.when(j > 0)
        def _():
            table_bytes_wait()

        return carry

    lax.fori_loop(0, N_CAT, per_table, 0)
    table_bytes_wait()

    pltpu.sync_copy(asm_v, out_hbm.at[:, pl.ds(base_b, B_PER_W)])


def kernel(x, tables):
    xt = x.T
    tt = jnp.transpose(tables, (0, 2, 1)).reshape(N_CAT * EMB * VOCAB)
    out_t = _emb_kernel(tt, xt)
    return out_t.T
